# Initial kernel scaffold; baseline (speedup 1.0000x reference)
import jax
import jax.numpy as jnp
from jax.experimental import pallas as pl


def kernel(inputs, routing_x):
    raise NotImplementedError("write your pallas kernel here")


# speedup vs baseline: 2.4355x; 2.4355x over previous
import functools

import jax
import jax.numpy as jnp
from jax import lax
from jax.experimental import pallas as pl
from jax.experimental.pallas import tpu as pltpu
from jax.experimental.pallas import tpu_sc as plsc

_NC = 2
_NS = 16
_LANES = 16


@functools.lru_cache(maxsize=None)
def _build(B, S, R, Wd):
    NW = _NC * _NS
    rows_pw = (B * S) // NW
    wpb = NW // B
    assert B * R == 2 * _LANES and rows_pw * wpb * B == B * S

    mesh = plsc.VectorSubcoreMesh(core_axis_name="c", subcore_axis_name="s")

    @functools.partial(
        pl.kernel,
        mesh=mesh,
        out_type=jax.ShapeDtypeStruct((B * S, Wd), jnp.float32),
        scratch_types=[
            pltpu.VMEM((B * R,), jnp.float32),
            pltpu.VMEM((rows_pw // 2, Wd + 64), jnp.float32),
            pltpu.VMEM((rows_pw // 2, Wd), jnp.float32),
            pltpu.SemaphoreType.DMA,
        ],
    )
    def run(tbl_hbm, routing_hbm, out_hbm, rout_v, win_v, rows_v, gsem):
        wid = lax.axis_index("s") * _NC + lax.axis_index("c")
        b = wid // wpb

        pltpu.sync_copy(routing_hbm, rout_v)
        per = _LANES // R
        vv = jnp.where(b < per, rout_v[pl.ds(0, _LANES)],
                       rout_v[pl.ds(_LANES, _LANES)])
        lanes = [vv[k] for k in range(_LANES)]
        lo = b % per
        logit = []
        for j in range(R):
            s = lanes[(per - 1) * R + j]
            for q in range(per - 2, -1, -1):
                s = jnp.where(lo == q, lanes[q * R + j], s)
            logit.append(s)
        best = logit[0]
        route = jnp.int32(0)
        for j in range(1, R):
            better = logit[j] > best
            route = jnp.where(better, jnp.int32(j), route)
            best = jnp.where(better, logit[j], best)

        shift = pl.multiple_of(64 * (route & 1), 64)
        col0 = pl.multiple_of(route * Wd - shift, 128)
        rowbase = wid * rows_pw
        half = rows_pw // 2
        for h in range(2):
            r0 = rowbase + h * half
            pltpu.async_copy(
                tbl_hbm.at[pl.ds(r0, half), pl.ds(col0, Wd + 64)],
                win_v, gsem).wait()

            def shift_row(i, _):
                for c in range(Wd // _LANES):
                    rows_v[i, pl.ds(c * _LANES, _LANES)] = (
                        win_v[i, pl.ds(shift + c * _LANES, _LANES)])
                return 0

            lax.fori_loop(0, half, shift_row, 0, unroll=2)
            pltpu.sync_copy(rows_v, out_hbm.at[pl.ds(r0, half)])

    return run


def kernel(inputs, routing_x):
    B, H, W, C = inputs.shape
    R = routing_x.shape[-1]
    Wd = C // R
    S = H * W
    tbl = inputs.reshape(B * S, C)
    out = _build(B, S, R, Wd)(tbl, routing_x.reshape(B * R))
    return out.reshape(B, H, W, Wd)


# speedup vs baseline: 2.9867x; 1.2263x over previous
import functools

import jax
import jax.numpy as jnp
from jax import lax
from jax.experimental import pallas as pl
from jax.experimental.pallas import tpu as pltpu
from jax.experimental.pallas import tpu_sc as plsc

_NC = 2
_NS = 16
_LANES = 16


@functools.lru_cache(maxsize=None)
def _build(B, S, R, Wd):
    NW = _NC * _NS
    rows_pw = (B * S) // NW
    wpb = NW // B
    nchunk = 4
    chunk = rows_pw // nchunk
    assert B * R == 2 * _LANES and rows_pw * wpb * B == B * S

    mesh = plsc.VectorSubcoreMesh(core_axis_name="c", subcore_axis_name="s")

    @functools.partial(
        pl.kernel,
        mesh=mesh,
        out_type=jax.ShapeDtypeStruct((B * S, Wd), jnp.float32),
        scratch_types=[
            pltpu.VMEM((B * R,), jnp.float32),
            pltpu.VMEM((2, chunk, Wd + 64), jnp.float32),
            pltpu.VMEM((2, chunk, Wd), jnp.float32),
            pltpu.SemaphoreType.DMA,
            pltpu.SemaphoreType.DMA,
            pltpu.SemaphoreType.DMA,
            pltpu.SemaphoreType.DMA,
        ],
    )
    def run(tbl_hbm, routing_hbm, out_hbm, rout_v, win_v, outb_v,
            g0, g1, w0, w1):
        gsems = (g0, g1)
        wsems = (w0, w1)
        wid = lax.axis_index("s") * _NC + lax.axis_index("c")
        b = wid // wpb

        pltpu.sync_copy(routing_hbm, rout_v)
        per = _LANES // R
        vv = jnp.where(b < per, rout_v[pl.ds(0, _LANES)],
                       rout_v[pl.ds(_LANES, _LANES)])
        lanes = [vv[k] for k in range(_LANES)]
        lo = b % per
        logit = []
        for j in range(R):
            s = lanes[(per - 1) * R + j]
            for q in range(per - 2, -1, -1):
                s = jnp.where(lo == q, lanes[q * R + j], s)
            logit.append(s)
        best = logit[0]
        route = jnp.int32(0)
        for j in range(1, R):
            better = logit[j] > best
            route = jnp.where(better, jnp.int32(j), route)
            best = jnp.where(better, logit[j], best)

        rowbase = wid * rows_pw
        sh = pl.multiple_of(64 * (route & 1), 64)
        col0 = pl.multiple_of(route * Wd - sh, 128)

        def in_copy(k):
            p = k % 2
            r0 = rowbase + k * chunk
            return pltpu.make_async_copy(
                tbl_hbm.at[pl.ds(r0, chunk), pl.ds(col0, Wd + 64)],
                win_v.at[p], gsems[p])

        def issue_in(k):
            in_copy(k).start()

        def wait_in(k):
            in_copy(k).wait()
            p = k % 2

            @plsc.parallel_loop(0, chunk, 1, unroll=4)
            def _shift(i):
                for c in range(Wd // _LANES):
                    outb_v[p, i, pl.ds(c * _LANES, _LANES)] = (
                        win_v[p, i, pl.ds(sh + c * _LANES, _LANES)])

        def out_copy(k):
            p = k % 2
            return pltpu.make_async_copy(
                outb_v.at[p],
                out_hbm.at[pl.ds(rowbase + k * chunk, chunk)], wsems[p])

        issue_in(0)
        issue_in(1)
        wait_in(0)
        out_copy(0).start()
        wait_in(1)
        out_copy(1).start()
        out_copy(0).wait()
        issue_in(2)
        wait_in(2)
        out_copy(2).start()
        out_copy(1).wait()
        issue_in(3)
        wait_in(3)
        out_copy(3).start()
        out_copy(2).wait()
        out_copy(3).wait()

    return run


def kernel(inputs, routing_x):
    B, H, W, C = inputs.shape
    R = routing_x.shape[-1]
    Wd = C // R
    S = H * W
    tbl = inputs.reshape(B * S, C)
    out = _build(B, S, R, Wd)(tbl, routing_x.reshape(B * R))
    return out.reshape(B, H, W, Wd)
